# baseline (speedup 1.0000x reference)
import jax
import jax.numpy as jnp
from jax.experimental import pallas as pl


def _gaines_div_kernel(d_ref, o_ref):
    o_ref[...] = (d_ref[0] + d_ref[1] > 0.0).astype(jnp.float32)


def kernel(dividend, divisor):
    del divisor
    _, rows, cols = dividend.shape
    block_rows = 1024
    grid = (rows // block_rows,)
    return pl.pallas_call(
        _gaines_div_kernel,
        grid=grid,
        in_specs=[pl.BlockSpec((2, block_rows, cols), lambda i: (0, i, 0))],
        out_specs=pl.BlockSpec((block_rows, cols), lambda i: (i, 0)),
        out_shape=jax.ShapeDtypeStruct((rows, cols), jnp.float32),
    )(dividend)


# speedup vs baseline: 1.0087x; 1.0087x over previous
import jax
import jax.numpy as jnp
from jax.experimental import pallas as pl


def _gaines_div_kernel(d_ref, o_ref):
    o_ref[...] = (d_ref[0] + d_ref[1] > 0.0).astype(jnp.float32)


def kernel(dividend, divisor):
    del divisor
    _, rows, cols = dividend.shape
    block_rows = 512
    grid = (rows // block_rows,)
    return pl.pallas_call(
        _gaines_div_kernel,
        grid=grid,
        in_specs=[pl.BlockSpec((2, block_rows, cols), lambda i: (0, i, 0))],
        out_specs=pl.BlockSpec((block_rows, cols), lambda i: (i, 0)),
        out_shape=jax.ShapeDtypeStruct((rows, cols), jnp.float32),
    )(dividend)


# speedup vs baseline: 1.0095x; 1.0007x over previous
import jax
import jax.numpy as jnp
from jax.experimental import pallas as pl


def _gaines_div_kernel(a_ref, b_ref, o_ref):
    o_ref[...] = (a_ref[...] + b_ref[...] > 0.0).astype(jnp.float32)


def kernel(dividend, divisor):
    del divisor
    _, rows, cols = dividend.shape
    flat = dividend.reshape(2 * rows, cols)
    block_rows = 512
    nblk = rows // block_rows
    off = nblk
    return pl.pallas_call(
        _gaines_div_kernel,
        grid=(nblk,),
        in_specs=[
            pl.BlockSpec((block_rows, cols), lambda i: (i, 0)),
            pl.BlockSpec((block_rows, cols), lambda i, o=off: (i + o, 0)),
        ],
        out_specs=pl.BlockSpec((block_rows, cols), lambda i: (i, 0)),
        out_shape=jax.ShapeDtypeStruct((rows, cols), jnp.float32),
    )(flat, flat)
